# baseline (speedup 1.0000x reference)
import jax
import jax.numpy as jnp
from jax import lax
from jax.experimental import pallas as pl
from jax.experimental.pallas import tpu as pltpu
from jax.experimental.pallas import tpu_sc as plsc

N = 65536
D = 64
NNZ = 4194304
LAYERS = 3

NC, NS = 2, 16
NW = NC * NS
EW = NNZ // NW
SUB = 8192
NSUB = EW // SUB
NB = 8
RB = N // NB
NBK = 16
CHUNK = 2048
UNIT = 128
NPAIR = CHUNK // (2 * UNIT)
CAP = EW + CHUNK
CAPR = CAP // UNIT
TOT = NW * NBK * CAP
TOTR = TOT // UNIT
PR = RB // 2
PSL = PR // NS

_params = pltpu.CompilerParams(needs_layout_passes=False)
_mesh = plsc.VectorSubcoreMesh(core_axis_name="c", subcore_axis_name="s")


def _iota16():
    return lax.broadcasted_iota(jnp.int32, (16,), 0)


def _bucketize_body(rows_hbm, cols_hbm, w_hbm,
                    brow_hbm, bcol_hbm, bw_hbm, cnt_hbm,
                    rbuf, cbuf, wbuf, rlbuf, dstbuf, cvbuf, padidx, sem):
    c = lax.axis_index("c")
    s = lax.axis_index("s")
    w = c * NS + s
    base_slot = w * (NBK * CAP)
    it = _iota16()

    def subchunk(t, cnts):
        off = w * EW + t * SUB
        pltpu.sync_copy(rows_hbm.at[pl.ds(off, SUB)], rbuf)
        pltpu.sync_copy(cols_hbm.at[pl.ds(off, SUB)], cbuf)
        pltpu.sync_copy(w_hbm.at[pl.ds(off, SUB)], wbuf)

        def vloop(j, cnts):
            cn = list(cnts)
            for k in range(8):
                v = j * 8 + k
                r = rbuf[pl.ds(v * 16, 16)]
                key = jnp.bitwise_or(
                    lax.shift_left(lax.shift_right_logical(r, 13), 1),
                    jnp.bitwise_and(r, 1))
                rlbuf[pl.ds(v * 16, 16)] = jnp.bitwise_and(r, RB - 1)
                pos = jnp.zeros((16,), jnp.int32)
                for b in range(NBK):
                    m = key == b
                    pref = plsc.cumsum(m.astype(jnp.int32))
                    pos = jnp.where(m, cn[b] + pref - 1, pos)
                    cn[b] = cn[b] + plsc.all_reduce_population_count(m)
                dst = base_slot + key * CAP + pos
                dj = dstbuf.at[j]
                dj[pl.ds(k * 16, 16)] = dst
            return tuple(cn)

        cnts = lax.fori_loop(0, SUB // 128, vloop, cnts)

        def sc_loop(jj, _):
            descs = []
            for j2 in range(4):
                idx = dstbuf.at[jj * 4 + j2]
                src0 = jj * 4 * UNIT + j2 * UNIT
                descs.append(pltpu.async_copy(
                    rlbuf.at[pl.ds(src0, UNIT)], brow_hbm.at[idx], sem))
                descs.append(pltpu.async_copy(
                    cbuf.at[pl.ds(src0, UNIT)], bcol_hbm.at[idx], sem))
                descs.append(pltpu.async_copy(
                    wbuf.at[pl.ds(src0, UNIT)], bw_hbm.at[idx], sem))
            for dsc in descs:
                dsc.wait()
            return 0

        lax.fori_loop(0, SUB // UNIT // 4, sc_loop, 0)
        return cnts

    zero8 = tuple(jnp.zeros((16,), jnp.int32) for _ in range(NBK))
    cnts = lax.fori_loop(0, NSUB, subchunk, zero8)

    for j in range(8):
        cbuf[pl.ds(j * 16, 16)] = jnp.zeros((16,), jnp.int32)
        wbuf[pl.ds(j * 16, 16)] = jnp.zeros((16,), jnp.float32)
    for kb in range(NBK):
        bb0 = base_slot + kb * CAP

        def pfill(j, _, _kb=kb, _bb0=bb0):
            pj = padidx.at[j]
            for k in range(8):
                pj[pl.ds(k * 16, 16)] = cnts[_kb] + (_bb0 + k * 16) + j * UNIT + it
            return 0

        lax.fori_loop(0, CHUNK // UNIT, pfill, 0)

        def pscat(jj, _):
            descs = []
            for j2 in range(4):
                idx = padidx.at[jj * 4 + j2]
                descs.append(pltpu.async_copy(
                    cbuf.at[pl.ds(0, UNIT)], brow_hbm.at[idx], sem))
                descs.append(pltpu.async_copy(
                    cbuf.at[pl.ds(0, UNIT)], bcol_hbm.at[idx], sem))
                descs.append(pltpu.async_copy(
                    wbuf.at[pl.ds(0, UNIT)], bw_hbm.at[idx], sem))
            for dsc in descs:
                dsc.wait()
            return 0

        lax.fori_loop(0, CHUNK // UNIT // 4, pscat, 0)

    cv = jnp.zeros((16,), jnp.int32)
    for b in range(NBK):
        cv = jnp.where(it == b, cnts[b], cv)
    cvbuf[...] = cv
    pltpu.sync_copy(cvbuf, cnt_hbm.at[w])


_bucketize = pl.kernel(
    _bucketize_body,
    out_type=(
        jax.ShapeDtypeStruct((TOT,), jnp.int32),
        jax.ShapeDtypeStruct((TOT,), jnp.int32),
        jax.ShapeDtypeStruct((TOT,), jnp.float32),
        jax.ShapeDtypeStruct((NW, 16), jnp.int32),
    ),
    mesh=_mesh,
    compiler_params=_params,
    scratch_types=[
        pltpu.VMEM((SUB,), jnp.int32),
        pltpu.VMEM((SUB,), jnp.int32),
        pltpu.VMEM((SUB,), jnp.float32),
        pltpu.VMEM((SUB,), jnp.int32),
        pltpu.VMEM((SUB // UNIT, UNIT), jnp.int32),
        pltpu.VMEM((16,), jnp.int32),
        pltpu.VMEM((CHUNK // UNIT, UNIT), jnp.int32),
        pltpu.SemaphoreType.DMA,
    ],
)


def _spmm_body(y_hbm, brow_hbm, bcol_hbm, bw_hbm, cnt_hbm, out_hbm,
               accum, zbuf, ebr, ebc, ebw, cvb, gix, six,
               gbuf0, gbuf1, pbuf0, pbuf1, gs0, gs1, ss0, ss1):
    c = lax.axis_index("c")
    s = lax.axis_index("s")
    it = _iota16()
    z16 = jnp.zeros((16,), jnp.float32)

    def zrow(e, _):
        zr = zbuf.at[e]
        for k in range(8):
            zr[pl.ds(k * 16, 16)] = z16
        return 0

    lax.fori_loop(0, UNIT, zrow, 0)

    def prow(e, _):
        p0 = pbuf0.at[e]
        p1 = pbuf1.at[e]
        for k in range(8):
            p0[pl.ds(k * 16, 16)] = z16
            p1[pl.ds(k * 16, 16)] = z16
        return 0

    def do_pair(up, par):
        hp = jnp.full((16,), par * 64, jnp.int32)
        bufs = ((gbuf0, pbuf0, gs0, ss0, 0), (gbuf1, pbuf1, gs1, ss1, 1))
        gds = []
        for half, (gb, _pb, gsem, _ssem, ix) in enumerate(bufs):
            u = up * 2 + half
            er = ebr.at[u]
            ec = ebc.at[u]
            gx = gix.at[ix]
            sx = six.at[ix]
            for k in range(8):
                sl = pl.ds(k * 16, 16)
                gx[sl] = lax.shift_right_logical(ec[sl], 1)
                sx[sl] = lax.shift_right_logical(er[sl], 1)
            gds.append(pltpu.async_copy(y_hbm.at[gix.at[ix]], gb, gsem))
        sds = []
        for half, (gb, pb, _gsem, ssem, ix) in enumerate(bufs):
            u = up * 2 + half
            gds[half].wait()

            def gscale(g, _, _u=u, _gb=gb, _pb=pb):
                sl = pl.ds(g * 16, 16)
                cp = jnp.bitwise_and(ebc.at[_u][sl], 1) * 64
                wv = ebw[pl.ds(_u * UNIT + g * 16, 16)]
                eids = g * 16 + it
                for d in range(D):
                    vals = plsc.load_gather(_gb, [eids, cp + d])
                    plsc.store_scatter(_pb, [eids, hp + d], vals * wv)
                return 0

            lax.fori_loop(0, UNIT // 16, gscale, 0)
            sds.append(pltpu.async_copy(
                pb, accum.at[six.at[ix]], ssem, add=True))
        sds[0].wait()
        sds[1].wait()
        return 0

    def block_pass(bb, _):
        b = c * (NB // NC) + bb
        for q in range(PSL // UNIT):
            pltpu.sync_copy(zbuf, accum.at[pl.ds(s * PSL + q * UNIT, UNIT)])
        plsc.subcore_barrier()

        def par_pass(par, _):
            bkt = b * 2 + par
            lax.fori_loop(0, UNIT, prow, 0)

            def h_pass(h, _):
                src_w = h * NS + s
                pltpu.sync_copy(cnt_hbm.at[src_w], cvb)
                cnt = jnp.max(jnp.where(it == bkt, cvb[...], 0))
                nch = lax.shift_right_logical(cnt + (CHUNK - 1), 11)
                bkt_row0 = (src_w * NBK + bkt) * CAPR
                bkt_w0 = (src_w * NBK + bkt) * CAP

                def chunk_body(i, _):
                    r0 = bkt_row0 + i * (CHUNK // UNIT)
                    pltpu.sync_copy(brow_hbm.at[pl.ds(r0, CHUNK // UNIT)], ebr)
                    pltpu.sync_copy(bcol_hbm.at[pl.ds(r0, CHUNK // UNIT)], ebc)
                    pltpu.sync_copy(bw_hbm.at[pl.ds(bkt_w0 + i * CHUNK, CHUNK)], ebw)
                    lax.fori_loop(
                        0, NPAIR,
                        lambda up, x: do_pair(up, par), 0)
                    return 0

                lax.fori_loop(0, nch, chunk_body, 0)
                return 0

            lax.fori_loop(0, 2, h_pass, 0)
            return 0

        lax.fori_loop(0, 2, par_pass, 0)
        plsc.subcore_barrier()
        for q in range(PSL // UNIT):
            row0 = b * PR + s * PSL + q * UNIT
            pltpu.sync_copy(accum.at[pl.ds(s * PSL + q * UNIT, UNIT)],
                            out_hbm.at[pl.ds(row0, UNIT)])
        plsc.subcore_barrier()
        return 0

    lax.fori_loop(0, NB // NC, block_pass, 0)


_spmm = pl.kernel(
    _spmm_body,
    out_type=jax.ShapeDtypeStruct((N // 2, 128), jnp.float32),
    mesh=_mesh,
    compiler_params=_params,
    scratch_types=[
        pltpu.VMEM_SHARED((PR, 128), jnp.float32),
        pltpu.VMEM((UNIT, 128), jnp.float32),
        pltpu.VMEM((CHUNK // UNIT, UNIT), jnp.int32),
        pltpu.VMEM((CHUNK // UNIT, UNIT), jnp.int32),
        pltpu.VMEM((CHUNK,), jnp.float32),
        pltpu.VMEM((16,), jnp.int32),
        pltpu.VMEM((2, UNIT), jnp.int32),
        pltpu.VMEM((2, UNIT), jnp.int32),
        pltpu.VMEM((UNIT, 128), jnp.float32),
        pltpu.VMEM((UNIT, 128), jnp.float32),
        pltpu.VMEM((UNIT, 128), jnp.float32),
        pltpu.VMEM((UNIT, 128), jnp.float32),
        pltpu.SemaphoreType.DMA,
        pltpu.SemaphoreType.DMA,
        pltpu.SemaphoreType.DMA,
        pltpu.SemaphoreType.DMA,
    ],
)


def kernel(x, rows, cols, weights):
    brow, bcol, bw, cnts = _bucketize(rows, cols, weights)
    brow2 = brow.reshape(TOTR, UNIT)
    bcol2 = bcol.reshape(TOTR, UNIT)
    y = x.reshape(N // 2, 128)
    for _ in range(LAYERS):
        y = _spmm(y, brow2, bcol2, bw, cnts)
    return y.reshape(N, D)


# speedup vs baseline: 1.0866x; 1.0866x over previous
import jax
import jax.numpy as jnp
from jax import lax
from jax.experimental import pallas as pl
from jax.experimental.pallas import tpu as pltpu
from jax.experimental.pallas import tpu_sc as plsc

N = 65536
D = 64
NNZ = 4194304
LAYERS = 3

NC, NS = 2, 16
NW = NC * NS
EW = NNZ // NW
SUB = 8192
NSUB = EW // SUB
NB = 8
RB = N // NB
NBK = 16
CHUNK = 2048
UNIT = 128
NPAIR = CHUNK // (2 * UNIT)
CAP = EW + CHUNK
CAPR = CAP // UNIT
TOT = NW * NBK * CAP
TOTR = TOT // UNIT
PR = RB // 2
PSL = PR // NS

_params = pltpu.CompilerParams(needs_layout_passes=False)
_mesh = plsc.VectorSubcoreMesh(core_axis_name="c", subcore_axis_name="s")


def _iota16():
    return lax.broadcasted_iota(jnp.int32, (16,), 0)


def _bucketize_body(rows_hbm, cols_hbm, w_hbm,
                    brow_hbm, bcol_hbm, bw_hbm, cnt_hbm,
                    rbuf, cbuf, wbuf, rlbuf, dstbuf, cvbuf, padidx, sem):
    c = lax.axis_index("c")
    s = lax.axis_index("s")
    w = c * NS + s
    base_slot = w * (NBK * CAP)
    it = _iota16()

    def subchunk(t, cnts):
        off = w * EW + t * SUB
        pltpu.sync_copy(rows_hbm.at[pl.ds(off, SUB)], rbuf)
        pltpu.sync_copy(cols_hbm.at[pl.ds(off, SUB)], cbuf)
        pltpu.sync_copy(w_hbm.at[pl.ds(off, SUB)], wbuf)

        def vloop(j, cnts):
            cn = list(cnts)
            for k in range(8):
                v = j * 8 + k
                r = rbuf[pl.ds(v * 16, 16)]
                key = jnp.bitwise_or(
                    lax.shift_left(lax.shift_right_logical(r, 13), 1),
                    jnp.bitwise_and(r, 1))
                rlbuf[pl.ds(v * 16, 16)] = jnp.bitwise_and(r, RB - 1)
                pos = jnp.zeros((16,), jnp.int32)
                for b in range(NBK):
                    m = key == b
                    pref = plsc.cumsum(m.astype(jnp.int32))
                    pos = jnp.where(m, cn[b] + pref - 1, pos)
                    cn[b] = cn[b] + plsc.all_reduce_population_count(m)
                dst = base_slot + key * CAP + pos
                dj = dstbuf.at[j]
                dj[pl.ds(k * 16, 16)] = dst
            return tuple(cn)

        cnts = lax.fori_loop(0, SUB // 128, vloop, cnts)

        def sc_loop(jj, _):
            descs = []
            for j2 in range(4):
                idx = dstbuf.at[jj * 4 + j2]
                src0 = jj * 4 * UNIT + j2 * UNIT
                descs.append(pltpu.async_copy(
                    rlbuf.at[pl.ds(src0, UNIT)], brow_hbm.at[idx], sem))
                descs.append(pltpu.async_copy(
                    cbuf.at[pl.ds(src0, UNIT)], bcol_hbm.at[idx], sem))
                descs.append(pltpu.async_copy(
                    wbuf.at[pl.ds(src0, UNIT)], bw_hbm.at[idx], sem))
            for dsc in descs:
                dsc.wait()
            return 0

        lax.fori_loop(0, SUB // UNIT // 4, sc_loop, 0)
        return cnts

    zero4 = tuple(jnp.zeros((16,), jnp.int32) for _ in range(NBK))
    cnts = lax.fori_loop(0, NSUB, subchunk, zero4)

    for j in range(8):
        cbuf[pl.ds(j * 16, 16)] = jnp.zeros((16,), jnp.int32)
        wbuf[pl.ds(j * 16, 16)] = jnp.zeros((16,), jnp.float32)
    for kb in range(NBK):
        bb0 = base_slot + kb * CAP

        def pfill(j, _, _kb=kb, _bb0=bb0):
            pj = padidx.at[j]
            for k in range(8):
                pj[pl.ds(k * 16, 16)] = cnts[_kb] + (_bb0 + k * 16) + j * UNIT + it
            return 0

        lax.fori_loop(0, CHUNK // UNIT, pfill, 0)

        def pscat(jj, _):
            descs = []
            for j2 in range(4):
                idx = padidx.at[jj * 4 + j2]
                descs.append(pltpu.async_copy(
                    cbuf.at[pl.ds(0, UNIT)], brow_hbm.at[idx], sem))
                descs.append(pltpu.async_copy(
                    cbuf.at[pl.ds(0, UNIT)], bcol_hbm.at[idx], sem))
                descs.append(pltpu.async_copy(
                    wbuf.at[pl.ds(0, UNIT)], bw_hbm.at[idx], sem))
            for dsc in descs:
                dsc.wait()
            return 0

        lax.fori_loop(0, CHUNK // UNIT // 4, pscat, 0)

    cv = jnp.zeros((16,), jnp.int32)
    for b in range(NBK):
        cv = jnp.where(it == b, cnts[b], cv)
    cvbuf[...] = cv
    pltpu.sync_copy(cvbuf, cnt_hbm.at[w])


_bucketize = pl.kernel(
    _bucketize_body,
    out_type=(
        jax.ShapeDtypeStruct((TOT,), jnp.int32),
        jax.ShapeDtypeStruct((TOT,), jnp.int32),
        jax.ShapeDtypeStruct((TOT,), jnp.float32),
        jax.ShapeDtypeStruct((NW, 16), jnp.int32),
    ),
    mesh=_mesh,
    compiler_params=_params,
    scratch_types=[
        pltpu.VMEM((SUB,), jnp.int32),
        pltpu.VMEM((SUB,), jnp.int32),
        pltpu.VMEM((SUB,), jnp.float32),
        pltpu.VMEM((SUB,), jnp.int32),
        pltpu.VMEM((SUB // UNIT, UNIT), jnp.int32),
        pltpu.VMEM((16,), jnp.int32),
        pltpu.VMEM((CHUNK // UNIT, UNIT), jnp.int32),
        pltpu.SemaphoreType.DMA,
    ],
)


def _spmm_body(y_hbm, brow_hbm, bcol_hbm, bw_hbm, cnt_hbm, out_hbm,
               accum, zbuf, ebr, ebc, ebw, cvb, gix, six,
               gbuf0, gbuf1, pbuf0, pbuf1, gs0, gs1, ss0, ss1):
    c = lax.axis_index("c")
    s = lax.axis_index("s")
    it = _iota16()
    z16 = jnp.zeros((16,), jnp.float32)
    zi16 = jnp.zeros((16,), jnp.int32)

    def zrow(e, _):
        zr = zbuf.at[e]
        for k in range(8):
            zr[pl.ds(k * 16, 16)] = z16
        return 0

    lax.fori_loop(0, UNIT, zrow, 0)

    def prow(e, _):
        p0 = pbuf0.at[e]
        p1 = pbuf1.at[e]
        for k in range(8):
            p0[pl.ds(k * 16, 16)] = z16
            p1[pl.ds(k * 16, 16)] = z16
        return 0

    def do_pair(up, par):
        hp = par * 64
        bufs = ((gbuf0, pbuf0, gs0, ss0, 0), (gbuf1, pbuf1, gs1, ss1, 1))
        gds = []
        for half, (gb, _pb, gsem, _ssem, ix) in enumerate(bufs):
            u = up * 2 + half
            er = ebr.at[u]
            ec = ebc.at[u]
            gx = gix.at[ix]
            sx = six.at[ix]
            for k in range(8):
                sl = pl.ds(k * 16, 16)
                gx[sl] = lax.shift_right_logical(ec[sl], 1)
                sx[sl] = lax.shift_right_logical(er[sl], 1)
            gds.append(pltpu.async_copy(y_hbm.at[gix.at[ix]], gb, gsem))
        sds = []
        for half, (gb, pb, _gsem, ssem, ix) in enumerate(bufs):
            u = up * 2 + half
            gds[half].wait()

            def gscale(g, _, _u=u, _gb=gb, _pb=pb):
                cpv = jnp.bitwise_and(ebc.at[_u][pl.ds(g * 16, 16)], 1)
                wv = ebw[pl.ds(_u * UNIT + g * 16, 16)]
                for e16 in range(16):
                    m = it == e16
                    lo = jnp.max(jnp.where(m, 1 - cpv, zi16)) == 1
                    ws = jnp.max(jnp.where(m, wv, z16))
                    e = g * 16 + e16
                    gr = _gb.at[e]
                    pr = _pb.at[e]
                    for k in range(4):
                        v = jnp.where(lo, gr[pl.ds(k * 16, 16)],
                                      gr[pl.ds(64 + k * 16, 16)])
                        pr[pl.ds(hp + k * 16, 16)] = v * ws
                return 0

            lax.fori_loop(0, UNIT // 16, gscale, 0)
            sds.append(pltpu.async_copy(
                pb, accum.at[six.at[ix]], ssem, add=True))
        sds[0].wait()
        sds[1].wait()
        return 0

    def block_pass(bb, _):
        b = c * (NB // NC) + bb
        for q in range(PSL // UNIT):
            pltpu.sync_copy(zbuf, accum.at[pl.ds(s * PSL + q * UNIT, UNIT)])
        plsc.subcore_barrier()

        for par in range(2):
            bkt = b * 2 + par
            lax.fori_loop(0, UNIT, prow, 0)

            def h_pass(h, _, bkt=bkt, par=par):
                src_w = h * NS + s
                pltpu.sync_copy(cnt_hbm.at[src_w], cvb)
                cnt = jnp.max(jnp.where(it == bkt, cvb[...], 0))
                nch = lax.shift_right_logical(cnt + (CHUNK - 1), 11)
                bkt_row0 = (src_w * NBK + bkt) * CAPR
                bkt_w0 = (src_w * NBK + bkt) * CAP

                def chunk_body(i, _, bkt_row0=bkt_row0, bkt_w0=bkt_w0, par=par):
                    r0 = bkt_row0 + i * (CHUNK // UNIT)
                    pltpu.sync_copy(brow_hbm.at[pl.ds(r0, CHUNK // UNIT)], ebr)
                    pltpu.sync_copy(bcol_hbm.at[pl.ds(r0, CHUNK // UNIT)], ebc)
                    pltpu.sync_copy(bw_hbm.at[pl.ds(bkt_w0 + i * CHUNK, CHUNK)], ebw)
                    lax.fori_loop(
                        0, NPAIR,
                        lambda up, x, par=par: do_pair(up, par), 0)
                    return 0

                lax.fori_loop(0, nch, chunk_body, 0)
                return 0

            lax.fori_loop(0, 2, h_pass, 0)

        plsc.subcore_barrier()
        for q in range(PSL // UNIT):
            row0 = b * PR + s * PSL + q * UNIT
            pltpu.sync_copy(accum.at[pl.ds(s * PSL + q * UNIT, UNIT)],
                            out_hbm.at[pl.ds(row0, UNIT)])
        plsc.subcore_barrier()
        return 0

    lax.fori_loop(0, NB // NC, block_pass, 0)


_spmm = pl.kernel(
    _spmm_body,
    out_type=jax.ShapeDtypeStruct((N // 2, 128), jnp.float32),
    mesh=_mesh,
    compiler_params=_params,
    scratch_types=[
        pltpu.VMEM_SHARED((PR, 128), jnp.float32),
        pltpu.VMEM((UNIT, 128), jnp.float32),
        pltpu.VMEM((CHUNK // UNIT, UNIT), jnp.int32),
        pltpu.VMEM((CHUNK // UNIT, UNIT), jnp.int32),
        pltpu.VMEM((CHUNK,), jnp.float32),
        pltpu.VMEM((16,), jnp.int32),
        pltpu.VMEM((2, UNIT), jnp.int32),
        pltpu.VMEM((2, UNIT), jnp.int32),
        pltpu.VMEM((UNIT, 128), jnp.float32),
        pltpu.VMEM((UNIT, 128), jnp.float32),
        pltpu.VMEM((UNIT, 128), jnp.float32),
        pltpu.VMEM((UNIT, 128), jnp.float32),
        pltpu.SemaphoreType.DMA,
        pltpu.SemaphoreType.DMA,
        pltpu.SemaphoreType.DMA,
        pltpu.SemaphoreType.DMA,
    ],
)


def kernel(x, rows, cols, weights):
    brow, bcol, bw, cnts = _bucketize(rows, cols, weights)
    brow2 = brow.reshape(TOTR, UNIT)
    bcol2 = bcol.reshape(TOTR, UNIT)
    y = x.reshape(N // 2, 128)
    for _ in range(LAYERS):
        y = _spmm(y, brow2, bcol2, bw, cnts)
    return y.reshape(N, D)
